# Initial kernel scaffold; baseline (speedup 1.0000x reference)
import jax
import jax.numpy as jnp
from jax.experimental import pallas as pl


def kernel(x, pos_emb):
    raise NotImplementedError("write your pallas kernel here")


# speedup vs baseline: 1.9470x; 1.9470x over previous
import jax
import jax.numpy as jnp
from jax.experimental import pallas as pl
from jax.experimental.pallas import tpu as pltpu

_TB = 512


def _add_kernel(x_ref, pe_ref, o_ref):
    o_ref[...] = x_ref[...] + pe_ref[...]


def kernel(x, pos_emb):
    B, T, C = x.shape
    grid = (T // _TB, B)
    return pl.pallas_call(
        _add_kernel,
        grid=grid,
        in_specs=[
            pl.BlockSpec((1, _TB, C), lambda t, b: (b, t, 0)),
            pl.BlockSpec((_TB, C), lambda t, b: (t, 0)),
        ],
        out_specs=pl.BlockSpec((1, _TB, C), lambda t, b: (b, t, 0)),
        out_shape=jax.ShapeDtypeStruct((B, T, C), x.dtype),
        compiler_params=pltpu.CompilerParams(
            dimension_semantics=("parallel", "arbitrary"),
        ),
    )(x, pos_emb)


# speedup vs baseline: 2.1079x; 1.0826x over previous
import jax
import jax.numpy as jnp
from jax.experimental import pallas as pl
from jax.experimental.pallas import tpu as pltpu

_TB = 1024


def _add_kernel(x_ref, pe_ref, o_ref):
    o_ref[...] = x_ref[...] + pe_ref[...]


def kernel(x, pos_emb):
    B, T, C = x.shape
    grid = (T // _TB, B)
    return pl.pallas_call(
        _add_kernel,
        grid=grid,
        in_specs=[
            pl.BlockSpec((1, _TB, C), lambda t, b: (b, t, 0)),
            pl.BlockSpec((_TB, C), lambda t, b: (t, 0)),
        ],
        out_specs=pl.BlockSpec((1, _TB, C), lambda t, b: (b, t, 0)),
        out_shape=jax.ShapeDtypeStruct((B, T, C), x.dtype),
        compiler_params=pltpu.CompilerParams(
            dimension_semantics=("parallel", "arbitrary"),
        ),
    )(x, pos_emb)


# speedup vs baseline: 2.2870x; 1.0850x over previous
import jax
import jax.numpy as jnp
from jax.experimental import pallas as pl
from jax.experimental.pallas import tpu as pltpu

_TB = 2048


def _add_kernel(x_ref, pe_ref, o_ref):
    o_ref[...] = x_ref[...] + pe_ref[...]


def kernel(x, pos_emb):
    B, T, C = x.shape
    grid = (T // _TB, B)
    return pl.pallas_call(
        _add_kernel,
        grid=grid,
        in_specs=[
            pl.BlockSpec((1, _TB, C), lambda t, b: (b, t, 0)),
            pl.BlockSpec((_TB, C), lambda t, b: (t, 0)),
        ],
        out_specs=pl.BlockSpec((1, _TB, C), lambda t, b: (b, t, 0)),
        out_shape=jax.ShapeDtypeStruct((B, T, C), x.dtype),
        compiler_params=pltpu.CompilerParams(
            dimension_semantics=("parallel", "arbitrary"),
        ),
    )(x, pos_emb)
